# baseline (speedup 1.0000x reference)
import functools

import jax
import jax.numpy as jnp
from jax import lax
from jax.experimental import pallas as pl
from jax.experimental.pallas import tpu as pltpu
from jax.experimental.pallas import tpu_sc as plsc

_D_IN = 8192
_D_F = 1024
_B = 4096
_NC = 2
_NS = 16
_NW = _NC * _NS
_RPW = _B // _NW
_L = 16
_GROUPS = _D_IN // _L

_mesh = plsc.VectorSubcoreMesh(core_axis_name="c", subcore_axis_name="s")


@functools.partial(
    pl.kernel,
    out_type=jax.ShapeDtypeStruct((_B, _D_F), jnp.float32),
    mesh=_mesh,
    compiler_params=pltpu.CompilerParams(needs_layout_passes=False),
    scratch_types=[
        pltpu.VMEM((_D_IN,), jnp.int32),
        pltpu.VMEM((_D_IN,), jnp.float32),
        pltpu.VMEM((_D_IN,), jnp.float32),
        pltpu.VMEM((_D_F,), jnp.float32),
    ],
)
def _countsketch(x_hbm, s_hbm, i_hbm, out_hbm, idx_v, sgn_v, x_v, acc_v):
    wid = lax.axis_index("s") * _NC + lax.axis_index("c")
    base = wid * _RPW
    pltpu.sync_copy(i_hbm, idx_v)
    pltpu.sync_copy(s_hbm, sgn_v)

    def row_body(r, carry):
        pltpu.sync_copy(x_hbm.at[base + r], x_v)

        def zero_body(k, c):
            acc_v[pl.ds(k * _L, _L)] = jnp.zeros((_L,), jnp.float32)
            return c

        lax.fori_loop(0, _D_F // _L, zero_body, 0)

        def scat_body(j, c):
            o = j * _L
            idx = idx_v[pl.ds(o, _L)]
            v = x_v[pl.ds(o, _L)] * sgn_v[pl.ds(o, _L)]
            plsc.addupdate_scatter(acc_v, [idx], v)
            return c

        lax.fori_loop(0, _GROUPS, scat_body, 0)
        pltpu.sync_copy(acc_v, out_hbm.at[base + r])
        return carry

    lax.fori_loop(0, _RPW, row_body, 0)


def kernel(x, s_hash, i_hash):
    return _countsketch(x, s_hash, i_hash.astype(jnp.int32))


# speedup vs baseline: 1.1771x; 1.1771x over previous
import functools

import jax
import jax.numpy as jnp
from jax import lax
from jax.experimental import pallas as pl
from jax.experimental.pallas import tpu as pltpu
from jax.experimental.pallas import tpu_sc as plsc

_D_IN = 8192
_D_F = 1024
_B = 4096
_NC = 2
_NS = 16
_NW = _NC * _NS
_RPW = _B // _NW
_L = 16
_GROUPS = _D_IN // _L
_RB = 8
_NBLK = _RPW // _RB

_mesh = plsc.VectorSubcoreMesh(core_axis_name="c", subcore_axis_name="s")


@functools.partial(
    pl.kernel,
    out_type=jax.ShapeDtypeStruct((_B, _D_F), jnp.float32),
    mesh=_mesh,
    compiler_params=pltpu.CompilerParams(needs_layout_passes=False),
    scratch_types=(
        [pltpu.VMEM((_D_IN,), jnp.int32),
         pltpu.VMEM((_D_IN,), jnp.float32)]
        + [pltpu.VMEM((_D_IN,), jnp.float32) for _ in range(_RB)]
        + [pltpu.VMEM((_D_F,), jnp.float32) for _ in range(_RB)]
    ),
)
def _countsketch(x_hbm, s_hbm, i_hbm, out_hbm, idx_v, sgn_v, *bufs):
    x_refs = bufs[:_RB]
    acc_refs = bufs[_RB:]
    wid = lax.axis_index("s") * _NC + lax.axis_index("c")
    base = wid * _RPW
    pltpu.sync_copy(i_hbm, idx_v)
    pltpu.sync_copy(s_hbm, sgn_v)

    def blk_body(bi, carry):
        row0 = base + bi * _RB
        for r in range(_RB):
            pltpu.sync_copy(x_hbm.at[row0 + r], x_refs[r])

        def zero_body(k, c):
            z = jnp.zeros((_L,), jnp.float32)
            for r in range(_RB):
                acc_refs[r][pl.ds(k * _L, _L)] = z
            return c

        lax.fori_loop(0, _D_F // _L, zero_body, 0)

        def scat_body(j, c):
            o = j * _L
            idx = idx_v[pl.ds(o, _L)]
            sgn = sgn_v[pl.ds(o, _L)]
            for r in range(_RB):
                v = x_refs[r][pl.ds(o, _L)] * sgn
                plsc.addupdate_scatter(acc_refs[r], [idx], v)
            return c

        lax.fori_loop(0, _GROUPS, scat_body, 0)
        for r in range(_RB):
            pltpu.sync_copy(acc_refs[r], out_hbm.at[row0 + r])
        return carry

    lax.fori_loop(0, _NBLK, blk_body, 0)


def kernel(x, s_hash, i_hash):
    return _countsketch(x, s_hash, i_hash.astype(jnp.int32))


# speedup vs baseline: 1.4807x; 1.2580x over previous
import functools

import jax
import jax.numpy as jnp
from jax import lax
from jax.experimental import pallas as pl
from jax.experimental.pallas import tpu as pltpu
from jax.experimental.pallas import tpu_sc as plsc

_D_IN = 8192
_D_F = 1024
_B = 4096
_NC = 2
_NS = 16
_NW = _NC * _NS
_RPW = _B // _NW
_L = 16
_GROUPS = _D_IN // _L
_RB = 4
_NBLK = _RPW // _RB

_mesh = plsc.VectorSubcoreMesh(core_axis_name="c", subcore_axis_name="s")


@functools.partial(
    pl.kernel,
    out_type=jax.ShapeDtypeStruct((_B, _D_F), jnp.float32),
    mesh=_mesh,
    compiler_params=pltpu.CompilerParams(needs_layout_passes=False),
    scratch_types=(
        [pltpu.VMEM((_D_IN,), jnp.int32),
         pltpu.VMEM((_D_IN,), jnp.float32)]
        + [pltpu.VMEM((_D_IN,), jnp.float32) for _ in range(2 * _RB)]
        + [pltpu.VMEM((_D_F,), jnp.float32) for _ in range(2 * _RB)]
        + [pltpu.SemaphoreType.DMA for _ in range(4)]
    ),
)
def _countsketch(x_hbm, s_hbm, i_hbm, out_hbm, idx_v, sgn_v, *bufs):
    x_refs = (bufs[0:_RB], bufs[_RB:2 * _RB])
    acc_refs = (bufs[2 * _RB:3 * _RB], bufs[3 * _RB:4 * _RB])
    sem_x = bufs[4 * _RB:4 * _RB + 2]
    sem_o = bufs[4 * _RB + 2:4 * _RB + 4]
    wid = lax.axis_index("s") * _NC + lax.axis_index("c")
    base = wid * _RPW
    pltpu.sync_copy(i_hbm, idx_v)
    pltpu.sync_copy(s_hbm, sgn_v)

    def start_x(bi, s):
        row0 = base + bi * _RB
        for r in range(_RB):
            pltpu.async_copy(x_hbm.at[row0 + r], x_refs[s][r], sem_x[s])

    def wait_x(s):
        for r in range(_RB):
            pltpu.make_async_copy(x_hbm.at[base], x_refs[s][r], sem_x[s]).wait()

    def start_out(bi, s):
        row0 = base + bi * _RB
        for r in range(_RB):
            pltpu.async_copy(acc_refs[s][r], out_hbm.at[row0 + r], sem_o[s])

    def wait_out(s):
        for r in range(_RB):
            pltpu.make_async_copy(acc_refs[s][r], out_hbm.at[base], sem_o[s]).wait()

    start_x(0, 0)

    def pair_body(p, carry):
        for par in (0, 1):
            bi = p * 2 + par
            nbi = jnp.minimum(bi + 1, _NBLK - 1)
            start_x(nbi, 1 - par)
            wait_x(par)

            @pl.when(bi >= 2)
            def _():
                wait_out(par)

            def zero_body(k, c):
                z = jnp.zeros((_L,), jnp.float32)
                for r in range(_RB):
                    acc_refs[par][r][pl.ds(k * _L, _L)] = z
                return c

            lax.fori_loop(0, _D_F // _L, zero_body, 0)

            def scat_body(j, c):
                o = j * _L
                idx = idx_v[pl.ds(o, _L)]
                sgn = sgn_v[pl.ds(o, _L)]
                for r in range(_RB):
                    v = x_refs[par][r][pl.ds(o, _L)] * sgn
                    plsc.addupdate_scatter(acc_refs[par][r], [idx], v)
                return c

            lax.fori_loop(0, _GROUPS, scat_body, 0)
            start_out(bi, par)
        return carry

    lax.fori_loop(0, _NBLK // 2, pair_body, 0)
    wait_x(0)
    wait_out(0)
    wait_out(1)


def kernel(x, s_hash, i_hash):
    return _countsketch(x, s_hash, i_hash.astype(jnp.int32))


# speedup vs baseline: 3.5252x; 2.3807x over previous
import functools

import jax
import jax.numpy as jnp
from jax import lax
from jax.experimental import pallas as pl
from jax.experimental.pallas import tpu as pltpu
from jax.experimental.pallas import tpu_sc as plsc

_D_IN = 8192
_D_F = 1024
_B = 4096
_NC = 2
_NS = 16
_NW = _NC * _NS
_RPW = _B // _NW
_L = 16
_GROUPS = _D_IN // _L
_RB = 4
_NBLK = _RPW // _RB

_mesh = plsc.VectorSubcoreMesh(core_axis_name="c", subcore_axis_name="s")


@functools.partial(
    pl.kernel,
    out_type=jax.ShapeDtypeStruct((_B, _D_F), jnp.float32),
    mesh=_mesh,
    compiler_params=pltpu.CompilerParams(needs_layout_passes=False),
    scratch_types=(
        [pltpu.VMEM((_D_IN,), jnp.int32),
         pltpu.VMEM((_D_IN,), jnp.float32)]
        + [pltpu.VMEM((_D_IN,), jnp.float32) for _ in range(2 * _RB)]
        + [pltpu.VMEM((_D_F,), jnp.float32) for _ in range(2 * _RB)]
        + [pltpu.SemaphoreType.DMA for _ in range(4)]
    ),
)
def _countsketch(x_hbm, s_hbm, i_hbm, out_hbm, idx_v, sgn_v, *bufs):
    x_refs = (bufs[0:_RB], bufs[_RB:2 * _RB])
    acc_refs = (bufs[2 * _RB:3 * _RB], bufs[3 * _RB:4 * _RB])
    sem_x = bufs[4 * _RB:4 * _RB + 2]
    sem_o = bufs[4 * _RB + 2:4 * _RB + 4]
    wid = lax.axis_index("s") * _NC + lax.axis_index("c")
    base = wid * _RPW
    pltpu.sync_copy(i_hbm, idx_v)
    pltpu.sync_copy(s_hbm, sgn_v)

    def start_x(bi, s):
        row0 = base + bi * _RB
        for r in range(_RB):
            pltpu.async_copy(x_hbm.at[row0 + r], x_refs[s][r], sem_x[s])

    def wait_x(s):
        for r in range(_RB):
            pltpu.make_async_copy(x_hbm.at[base], x_refs[s][r], sem_x[s]).wait()

    def start_out(bi, s):
        row0 = base + bi * _RB
        for r in range(_RB):
            pltpu.async_copy(acc_refs[s][r], out_hbm.at[row0 + r], sem_o[s])

    def wait_out(s):
        for r in range(_RB):
            pltpu.make_async_copy(acc_refs[s][r], out_hbm.at[base], sem_o[s]).wait()

    start_x(0, 0)

    def pair_body(p, carry):
        for par in (0, 1):
            bi = p * 2 + par
            nbi = jnp.minimum(bi + 1, _NBLK - 1)
            start_x(nbi, 1 - par)
            wait_x(par)

            @pl.when(bi >= 2)
            def _():
                wait_out(par)

            @plsc.parallel_loop(0, _D_F // _L, unroll=4)
            def zero_body(k):
                z = jnp.zeros((_L,), jnp.float32)
                for r in range(_RB):
                    acc_refs[par][r][pl.ds(k * _L, _L)] = z

            @plsc.parallel_loop(0, _GROUPS, unroll=4)
            def scat_body(j):
                o = j * _L
                idx = idx_v[pl.ds(o, _L)]
                sgn = sgn_v[pl.ds(o, _L)]
                for r in range(_RB):
                    v = x_refs[par][r][pl.ds(o, _L)] * sgn
                    plsc.addupdate_scatter(acc_refs[par][r], [idx], v)
            start_out(bi, par)
        return carry

    lax.fori_loop(0, _NBLK // 2, pair_body, 0)
    wait_x(0)
    wait_out(0)
    wait_out(1)


def kernel(x, s_hash, i_hash):
    return _countsketch(x, s_hash, i_hash.astype(jnp.int32))
